# Initial kernel scaffold; baseline (speedup 1.0000x reference)
import jax
import jax.numpy as jnp
from jax.experimental import pallas as pl


def kernel(content_title, content_description, topic_title, topic_description, E_ct, E_cd, E_tt, E_td, W1, b1, W2, b2, W3, b3):
    raise NotImplementedError("write your pallas kernel here")


# speedup vs baseline: 17.1815x; 17.1815x over previous
import functools

import jax
import jax.numpy as jnp
from jax import lax
from jax.experimental import pallas as pl
from jax.experimental.pallas import tpu as pltpu
from jax.experimental.pallas import tpu_sc as plsc

B = 16384
LT = 20
LD = 200
EMB = 30
FEAT = 4 * EMB

NC = 2
NS = 16
NW = NC * NS
ROWS_PER_W = B // NW
NB = 16
CHUNKS = ROWS_PER_W // NB


GT = 64
GD = 128
KT = NB * LT // GT
KD = NB * LD // GD


def _pool_body(ct_h, cd_h, tt_h, td_h, e_ct, e_cd, e_tt, e_td, out_h,
               idx_ct, idx_cd, idx_tt, idx_td, rows, acc, sem):
    wid = lax.axis_index("s") * NC + lax.axis_index("c")

    def chunk_body(ci, carry):
        base = wid * ROWS_PER_W + ci * NB
        pltpu.sync_copy(ct_h.at[pl.ds(base * LT // GT, KT), :], idx_ct)
        pltpu.sync_copy(cd_h.at[pl.ds(base * LD // GD, KD), :], idx_cd)
        pltpu.sync_copy(tt_h.at[pl.ds(base * LT // GT, KT), :], idx_tt)
        pltpu.sync_copy(td_h.at[pl.ds(base * LD // GD, KD), :], idx_td)

        for idx_v, seq_len, gsz, e_h, col in (
            (idx_ct, LT, GT, e_ct, 0),
            (idx_cd, LD, GD, e_cd, EMB),
            (idx_tt, LT, GT, e_tt, 2 * EMB),
            (idx_td, LD, GD, e_td, 3 * EMB),
        ):
            k = NB * seq_len // gsz
            copies = [
                pltpu.async_copy(e_h.at[idx_v.at[j]],
                                 rows.at[pl.ds(j * gsz, gsz)], sem)
                for j in range(k)
            ]
            for c in copies:
                c.wait()
            for b in range(NB):
                r0 = b * seq_len

                def red(r, c, r0=r0):
                    a0, a1 = c
                    a0 = a0 + rows[r0 + r, pl.ds(0, 16)]
                    a1 = a1 + rows[r0 + r, pl.ds(EMB - 16, 16)]
                    return a0, a1

                zero = jnp.zeros((16,), jnp.float32)
                a0, a1 = lax.fori_loop(0, seq_len, red, (zero, zero))
                acc[b, pl.ds(col, 16)] = a0
                acc[b, pl.ds(col + EMB - 16, 16)] = a1

        pltpu.sync_copy(acc, out_h.at[pl.ds(base, NB), :])
        return carry

    lax.fori_loop(0, CHUNKS, chunk_body, 0)


_pool = functools.partial(
    pl.kernel,
    out_type=jax.ShapeDtypeStruct((B, FEAT), jnp.float32),
    mesh=plsc.VectorSubcoreMesh(core_axis_name="c", subcore_axis_name="s"),
    compiler_params=pltpu.CompilerParams(use_tc_tiling_on_sc=False),
    scratch_types=[
        pltpu.VMEM((KT, GT), jnp.int32),
        pltpu.VMEM((KD, GD), jnp.int32),
        pltpu.VMEM((KT, GT), jnp.int32),
        pltpu.VMEM((KD, GD), jnp.int32),
        pltpu.VMEM((NB * LD, EMB), jnp.float32),
        pltpu.VMEM((NB, FEAT), jnp.float32),
        pltpu.SemaphoreType.DMA,
    ],
)(_pool_body)


BM = 1024


def _mlp_body(x_ref, w1_ref, b1_ref, w2_ref, b2_ref, w3_ref, b3_ref, o_ref):
    h = jnp.maximum(x_ref[...], 0.0)
    h = jnp.maximum(
        jnp.dot(h, w1_ref[...], preferred_element_type=jnp.float32)
        + b1_ref[...], 0.0)
    h = jnp.maximum(
        jnp.dot(h, w2_ref[...], preferred_element_type=jnp.float32)
        + b2_ref[...], 0.0)
    z = jnp.dot(h, w3_ref[...], preferred_element_type=jnp.float32) + b3_ref[...]
    o_ref[...] = 1.0 / (1.0 + jnp.exp(-z))


_mlp = pl.pallas_call(
    _mlp_body,
    grid=(B // BM,),
    in_specs=[
        pl.BlockSpec((BM, FEAT), lambda i: (i, 0)),
        pl.BlockSpec((FEAT, EMB), lambda i: (0, 0)),
        pl.BlockSpec((1, EMB), lambda i: (0, 0)),
        pl.BlockSpec((EMB, EMB), lambda i: (0, 0)),
        pl.BlockSpec((1, EMB), lambda i: (0, 0)),
        pl.BlockSpec((EMB, 1), lambda i: (0, 0)),
        pl.BlockSpec((1, 1), lambda i: (0, 0)),
    ],
    out_specs=pl.BlockSpec((BM, 1), lambda i: (i, 0)),
    out_shape=jax.ShapeDtypeStruct((B, 1), jnp.float32),
)


def kernel(content_title, content_description, topic_title, topic_description,
           E_ct, E_cd, E_tt, E_td, W1, b1, W2, b2, W3, b3):
    ct = content_title.astype(jnp.int32).reshape(-1, GT)
    cd = content_description.astype(jnp.int32).reshape(-1, GD)
    tt = topic_title.astype(jnp.int32).reshape(-1, GT)
    td = topic_description.astype(jnp.int32).reshape(-1, GD)
    pooled = _pool(ct, cd, tt, td, E_ct, E_cd, E_tt, E_td)
    return _mlp(pooled, W1, b1.reshape(1, EMB), W2, b2.reshape(1, EMB),
                W3, b3.reshape(1, 1))
